# baseline (speedup 1.0000x reference)
import functools

import jax
import jax.numpy as jnp
from jax import lax
from jax.experimental import pallas as pl
from jax.experimental.pallas import tpu as pltpu
from jax.experimental.pallas import tpu_sc as plsc

VOCAB = 50257
D = 768
B = 4
S = 2048
N = S * B
NC = 2
NS = 16
NW = NC * NS
RPW = N // NW
SPW = S // NW
NCH = 8
C = RPW // NCH
CS = C // B
LANES = 16
NDB = D // LANES
NBUF = 4
LOOK = 2


def _sc_embed(idx3, wte, wpe):
    mesh = plsc.VectorSubcoreMesh(core_axis_name="c", subcore_axis_name="s")

    @functools.partial(
        pl.kernel,
        mesh=mesh,
        out_type=jax.ShapeDtypeStruct((S, B, D), jnp.float32),
        scratch_types=(
            [pltpu.VMEM((NCH, C), jnp.int32)]
            + [pltpu.VMEM((C, D), jnp.float32)] * NBUF
            + [pltpu.VMEM((CS, D), jnp.float32)] * 2
            + [pltpu.SemaphoreType.DMA] * (NBUF + NBUF + 2)
        ),
    )
    def k(idx_hbm, wte_hbm, wpe_hbm, out_hbm,
          idx_v, r0, r1, r2, r3, p0, p1,
          g0, g1, g2, g3, w0, w1, w2, w3, q0, q1):
        bufs = (r0, r1, r2, r3)
        pes = (p0, p1)
        gsems = (g0, g1, g2, g3)
        wsems = (w0, w1, w2, w3)
        qsems = (q0, q1)
        wid = lax.axis_index("s") * NC + lax.axis_index("c")
        s0 = wid * SPW
        pltpu.sync_copy(idx_hbm.at[wid], idx_v)
        gdescs = [None] * NBUF
        pdescs = [None] * 2

        def drain_writes(buf, sem):
            for _ in range(CS):
                pltpu.make_async_copy(
                    buf.at[pl.ds(0, B)], out_hbm.at[0], sem).wait()

        for j in range(2):
            pdescs[j] = pltpu.async_copy(
                wpe_hbm.at[pl.ds(s0 + j * CS, CS)], pes[j], qsems[j])
        for j in range(LOOK):
            gdescs[j] = pltpu.async_copy(
                wte_hbm.at[idx_v.at[j]], bufs[j], gsems[j])
        for j in range(NCH):
            b = j % NBUF
            jl = j + LOOK
            if jl < NCH:
                nb = jl % NBUF
                if j >= LOOK:
                    drain_writes(bufs[nb], wsems[nb])
                gdescs[nb] = pltpu.async_copy(
                    wte_hbm.at[idx_v.at[jl]], bufs[nb], gsems[nb])
            if j + 1 < NCH and j >= 1:
                pdescs[(j + 1) % 2] = pltpu.async_copy(
                    wpe_hbm.at[pl.ds(s0 + (j + 1) * CS, CS)],
                    pes[(j + 1) % 2], qsems[(j + 1) % 2])
            gdescs[b].wait()
            pdescs[j % 2].wait()
            buf = bufs[b]
            pe = pes[j % 2]

            def pos_body(sl, carry, buf=buf, pe=pe, j=j, b=b):
                r0_ = sl * B
                for db in range(NDB):
                    off = db * LANES
                    w = pe[sl, pl.ds(off, LANES)]
                    for bb in range(B):
                        buf[r0_ + bb, pl.ds(off, LANES)] = (
                            buf[r0_ + bb, pl.ds(off, LANES)] + w
                        )
                pltpu.async_copy(
                    buf.at[pl.ds(r0_, B)],
                    out_hbm.at[s0 + j * CS + sl],
                    wsems[b])
                return carry

            lax.fori_loop(0, CS, pos_body, 0)
        for j in range(NCH - NBUF, NCH):
            drain_writes(bufs[j % NBUF], wsems[j % NBUF])

    return k(idx3, wte, wpe)


def kernel(input_ids, wte, wpe):
    idx3 = jnp.transpose(input_ids).reshape(NW, NCH, C)
    hidden = _sc_embed(idx3, wte, wpe)
    return (hidden, input_ids)


# speedup vs baseline: 1.0756x; 1.0756x over previous
import functools

import jax
import jax.numpy as jnp
from jax import lax
from jax.experimental import pallas as pl
from jax.experimental.pallas import tpu as pltpu
from jax.experimental.pallas import tpu_sc as plsc

VOCAB = 50257
D = 768
B = 4
S = 2048
N = S * B
NC = 2
NS = 16
NW = NC * NS
RPW = N // NW
SPW = S // NW
NCH = 16
C = RPW // NCH
CS = C // B
LANES = 16
NDB = D // LANES
NBUF = 4
LOOK = 2


def _sc_embed(idx3, wte, wpe):
    mesh = plsc.VectorSubcoreMesh(core_axis_name="c", subcore_axis_name="s")

    @functools.partial(
        pl.kernel,
        mesh=mesh,
        out_type=jax.ShapeDtypeStruct((S, B, D), jnp.float32),
        scratch_types=(
            [pltpu.VMEM((NCH, C), jnp.int32)]
            + [pltpu.VMEM((C, D), jnp.float32)] * NBUF
            + [pltpu.VMEM((CS, D), jnp.float32)] * 2
            + [pltpu.SemaphoreType.DMA] * (NBUF + NBUF + 2)
        ),
    )
    def k(idx_hbm, wte_hbm, wpe_hbm, out_hbm,
          idx_v, r0, r1, r2, r3, p0, p1,
          g0, g1, g2, g3, w0, w1, w2, w3, q0, q1):
        bufs = (r0, r1, r2, r3)
        pes = (p0, p1)
        gsems = (g0, g1, g2, g3)
        wsems = (w0, w1, w2, w3)
        qsems = (q0, q1)
        wid = lax.axis_index("s") * NC + lax.axis_index("c")
        s0 = wid * SPW
        pltpu.sync_copy(idx_hbm.at[wid], idx_v)
        gdescs = [None] * NBUF
        wdescs = [None] * NBUF
        pdescs = [None] * 2
        for j in range(2):
            pdescs[j] = pltpu.async_copy(
                wpe_hbm.at[pl.ds(s0 + j * CS, CS)], pes[j], qsems[j])
        for j in range(LOOK):
            gdescs[j] = pltpu.async_copy(
                wte_hbm.at[idx_v.at[j]], bufs[j], gsems[j])
        for j in range(NCH):
            b = j % NBUF
            jl = j + LOOK
            if jl < NCH:
                nb = jl % NBUF
                if j >= LOOK:
                    for d in wdescs[nb]:
                        d.wait()
                gdescs[nb] = pltpu.async_copy(
                    wte_hbm.at[idx_v.at[jl]], bufs[nb], gsems[nb])
            if j + 1 < NCH and j >= 1:
                pdescs[(j + 1) % 2] = pltpu.async_copy(
                    wpe_hbm.at[pl.ds(s0 + (j + 1) * CS, CS)],
                    pes[(j + 1) % 2], qsems[(j + 1) % 2])
            gdescs[b].wait()
            pdescs[j % 2].wait()
            buf = bufs[b]
            pe = pes[j % 2]

            def add_body(i, carry, buf=buf, pe=pe):
                off = i * LANES
                for sl in range(CS):
                    w = pe[sl, pl.ds(off, LANES)]
                    for bb in range(B):
                        r = sl * B + bb
                        buf[r, pl.ds(off, LANES)] = (
                            buf[r, pl.ds(off, LANES)] + w
                        )
                return carry

            lax.fori_loop(0, NDB, add_body, 0)
            ds = []
            for sl in range(CS):
                ds.append(pltpu.async_copy(
                    buf.at[pl.ds(sl * B, B)],
                    out_hbm.at[s0 + j * CS + sl],
                    wsems[b]))
            wdescs[b] = ds
        for j in range(NCH - NBUF, NCH):
            if wdescs[j % NBUF] is not None:
                for d in wdescs[j % NBUF]:
                    d.wait()

    return k(idx3, wte, wpe)


def kernel(input_ids, wte, wpe):
    idx3 = jnp.transpose(input_ids).reshape(NW, NCH, C)
    hidden = _sc_embed(idx3, wte, wpe)
    return (hidden, input_ids)


# speedup vs baseline: 1.1348x; 1.0551x over previous
import functools

import jax
import jax.numpy as jnp
from jax import lax
from jax.experimental import pallas as pl
from jax.experimental.pallas import tpu as pltpu
from jax.experimental.pallas import tpu_sc as plsc

VOCAB = 50257
D = 768
B = 4
S = 2048
N = S * B
NC = 2
NS = 16
NW = NC * NS
RPW = N // NW
SPW = S // NW
NCH = 8
C = RPW // NCH
CS = C // B
LANES = 16
NDB = D // LANES
NBUF = 4
LOOK = 2


def _sc_embed(idx3, wte, wpe):
    mesh = plsc.VectorSubcoreMesh(core_axis_name="c", subcore_axis_name="s")

    @functools.partial(
        pl.kernel,
        mesh=mesh,
        out_type=jax.ShapeDtypeStruct((S, B, D), jnp.float32),
        scratch_types=(
            [pltpu.VMEM((NCH, C), jnp.int32)]
            + [pltpu.VMEM((C, D), jnp.float32)] * NBUF
            + [pltpu.VMEM((CS, D), jnp.float32)] * 2
            + [pltpu.SemaphoreType.DMA] * (NBUF + NBUF + 2)
        ),
    )
    def k(idx_hbm, wte_hbm, wpe_hbm, out_hbm,
          idx_v, r0, r1, r2, r3, p0, p1,
          g0, g1, g2, g3, w0, w1, w2, w3, q0, q1):
        bufs = (r0, r1, r2, r3)
        pes = (p0, p1)
        gsems = (g0, g1, g2, g3)
        wsems = (w0, w1, w2, w3)
        qsems = (q0, q1)
        wid = lax.axis_index("s") * NC + lax.axis_index("c")
        s0 = wid * SPW
        pltpu.sync_copy(idx_hbm.at[wid], idx_v)
        gdescs = [None] * NBUF
        wdescs = [None] * NBUF
        pdescs = [None] * 2
        for j in range(2):
            pdescs[j] = pltpu.async_copy(
                wpe_hbm.at[pl.ds(s0 + j * CS, CS)], pes[j], qsems[j])
        for j in range(LOOK):
            gdescs[j] = pltpu.async_copy(
                wte_hbm.at[idx_v.at[j]], bufs[j], gsems[j])
        for j in range(NCH):
            b = j % NBUF
            jl = j + LOOK
            if jl < NCH:
                nb = jl % NBUF
                if j >= LOOK:
                    for d in wdescs[nb]:
                        d.wait()
                gdescs[nb] = pltpu.async_copy(
                    wte_hbm.at[idx_v.at[jl]], bufs[nb], gsems[nb])
            if j + 1 < NCH and j >= 1:
                pdescs[(j + 1) % 2] = pltpu.async_copy(
                    wpe_hbm.at[pl.ds(s0 + (j + 1) * CS, CS)],
                    pes[(j + 1) % 2], qsems[(j + 1) % 2])
            gdescs[b].wait()
            pdescs[j % 2].wait()
            buf = bufs[b]
            pe = pes[j % 2]

            def add_body(i, carry, buf=buf, pe=pe):
                off = i * LANES
                for sl in range(CS):
                    w = pe[sl, pl.ds(off, LANES)]
                    for bb in range(B):
                        r = sl * B + bb
                        buf[r, pl.ds(off, LANES)] = (
                            buf[r, pl.ds(off, LANES)] + w
                        )
                return carry

            lax.fori_loop(0, NDB, add_body, 0)
            ds = []
            for sl in range(CS):
                ds.append(pltpu.async_copy(
                    buf.at[pl.ds(sl * B, B)],
                    out_hbm.at[s0 + j * CS + sl],
                    wsems[b]))
            wdescs[b] = ds
        for j in range(NCH - NBUF, NCH):
            if wdescs[j % NBUF] is not None:
                for d in wdescs[j % NBUF]:
                    d.wait()

    return k(idx3, wte, wpe)


def kernel(input_ids, wte, wpe):
    idx3 = jnp.transpose(input_ids).reshape(NW, NCH, C)
    hidden = _sc_embed(idx3, wte, wpe)
    return (hidden, input_ids)


# speedup vs baseline: 1.1556x; 1.0183x over previous
import functools

import jax
import jax.numpy as jnp
from jax import lax
from jax.experimental import pallas as pl
from jax.experimental.pallas import tpu as pltpu
from jax.experimental.pallas import tpu_sc as plsc

VOCAB = 50257
D = 768
B = 4
S = 2048
N = S * B
NC = 2
NS = 16
NW = NC * NS
RPW = N // NW
SPW = S // NW
NCH = 8
C = RPW // NCH
CS = C // B
LANES = 16
NDB = D // LANES
NBUF = 4
LOOK = 2


def _sc_embed(idx3, wte, wpe):
    mesh = plsc.VectorSubcoreMesh(core_axis_name="c", subcore_axis_name="s")

    @functools.partial(
        pl.kernel,
        mesh=mesh,
        out_type=jax.ShapeDtypeStruct((S, B, D), jnp.float32),
        scratch_types=(
            [pltpu.VMEM((NCH, C), jnp.int32)]
            + [pltpu.VMEM((C, D), jnp.float32)] * NBUF
            + [pltpu.VMEM((CS, D), jnp.float32)] * 2
            + [pltpu.SemaphoreType.DMA] * (NBUF + NBUF + 2)
        ),
    )
    def k(idx_hbm, wte_hbm, wpe_hbm, out_hbm,
          idx_v, r0, r1, r2, r3, p0, p1,
          g0, g1, g2, g3, w0, w1, w2, w3, q0, q1):
        bufs = (r0, r1, r2, r3)
        pes = (p0, p1)
        gsems = (g0, g1, g2, g3)
        wsems = (w0, w1, w2, w3)
        qsems = (q0, q1)
        wid = lax.axis_index("s") * NC + lax.axis_index("c")
        s0 = wid * SPW
        pltpu.sync_copy(idx_hbm.at[wid, pl.ds(0, LOOK)],
                        idx_v.at[pl.ds(0, LOOK)])
        gdescs = [None] * NBUF
        wdescs = [None] * NBUF
        pdescs = [None] * 2
        for j in range(2):
            pdescs[j] = pltpu.async_copy(
                wpe_hbm.at[pl.ds(s0 + j * CS, CS)], pes[j], qsems[j])
        for j in range(LOOK):
            gdescs[j] = pltpu.async_copy(
                wte_hbm.at[idx_v.at[j]], bufs[j], gsems[j])
        pltpu.sync_copy(idx_hbm.at[wid, pl.ds(LOOK, NCH - LOOK)],
                        idx_v.at[pl.ds(LOOK, NCH - LOOK)])
        for j in range(NCH):
            b = j % NBUF
            jl = j + LOOK
            if jl < NCH:
                nb = jl % NBUF
                if j >= LOOK:
                    for d in wdescs[nb]:
                        d.wait()
                gdescs[nb] = pltpu.async_copy(
                    wte_hbm.at[idx_v.at[jl]], bufs[nb], gsems[nb])
            if j + 1 < NCH and j >= 1:
                pdescs[(j + 1) % 2] = pltpu.async_copy(
                    wpe_hbm.at[pl.ds(s0 + (j + 1) * CS, CS)],
                    pes[(j + 1) % 2], qsems[(j + 1) % 2])
            gdescs[b].wait()
            pdescs[j % 2].wait()
            buf = bufs[b]
            pe = pes[j % 2]

            def add_body(i, carry, buf=buf, pe=pe):
                off = i * LANES
                for sl in range(CS):
                    w = pe[sl, pl.ds(off, LANES)]
                    for bb in range(B):
                        r = sl * B + bb
                        buf[r, pl.ds(off, LANES)] = (
                            buf[r, pl.ds(off, LANES)] + w
                        )
                return carry

            lax.fori_loop(0, NDB, add_body, 0)
            ds = []
            for sl in range(CS):
                ds.append(pltpu.async_copy(
                    buf.at[pl.ds(sl * B, B)],
                    out_hbm.at[s0 + j * CS + sl],
                    wsems[b]))
            wdescs[b] = ds
        for j in range(NCH - NBUF, NCH):
            if wdescs[j % NBUF] is not None:
                for d in wdescs[j % NBUF]:
                    d.wait()

    return k(idx3, wte, wpe)


def kernel(input_ids, wte, wpe):
    idx3 = jnp.transpose(input_ids).reshape(NW, NCH, C)
    hidden = _sc_embed(idx3, wte, wpe)
    return (hidden, input_ids)
